# baseline (speedup 1.0000x reference)
import functools

import jax
import jax.numpy as jnp
from jax import lax
from jax.experimental import pallas as pl
from jax.experimental.pallas import tpu as pltpu
from jax.experimental.pallas import tpu_sc as plsc

N = 50000
E = 800000
H = 128
NP = 50048
RB = 2176
NBLK = NP // RB
NPT = NP // 16
G = E // 128
Q1, R1 = divmod(G, 32)
Q2, R2 = divmod(G, 16)
NT = NP // 128
QF, RF = divmod(NT, 16)

_mesh = plsc.VectorSubcoreMesh(core_axis_name="c", subcore_axis_name="s",
                               num_cores=2, num_subcores=16)


def _leaky(v, slope):
    return jnp.where(v >= 0, v, v * slope)


def _interleave_halves(blkc):
    r = blkc.shape[0]
    return jnp.reshape(jnp.stack([blkc[:, :16], blkc[:, 16:]], axis=-1),
                       (r, 32))


def _tc_pre_body(x_ref, w_ref, ats_ref, atd_ref,
                 xpt_ref, xptb_ref, as_ref, ad_ref, m_ref, scr):
    i = pl.program_id(0)
    xp = jnp.dot(x_ref[...], w_ref[...], preferred_element_type=jnp.float32)
    a_s = jnp.sum(xp * ats_ref[...], axis=1)
    a_d = jnp.sum(xp * atd_ref[...], axis=1)
    as_ref[...] = a_s[None]
    ad_ref[...] = a_d[None]
    for b in range(4):
        blkc = xp[:, b * 32:(b + 1) * 32]
        xpt_ref[b] = blkc
        xptb_ref[b] = _interleave_halves(blkc).astype(jnp.bfloat16)
    ms = jnp.max(a_s)
    md = jnp.max(a_d)

    @pl.when(i == 0)
    def _():
        scr[0] = ms
        scr[1] = md

    @pl.when(i > 0)
    def _():
        scr[0] = jnp.maximum(scr[0], ms)
        scr[1] = jnp.maximum(scr[1], md)

    m_ref[...] = jnp.full((1, 128), _leaky(scr[0] + scr[1], 0.2), jnp.float32)


def _tc_pre(xpad, w, ats, atd):
    f = xpad.shape[1]
    return pl.pallas_call(
        _tc_pre_body,
        grid=(NBLK,),
        in_specs=[
            pl.BlockSpec((RB, f), lambda i: (i, 0)),
            pl.BlockSpec((f, H), lambda i: (0, 0)),
            pl.BlockSpec((1, H), lambda i: (0, 0)),
            pl.BlockSpec((1, H), lambda i: (0, 0)),
        ],
        out_specs=[
            pl.BlockSpec((4, RB, 32), lambda i: (0, i, 0)),
            pl.BlockSpec((4, RB, 32), lambda i: (0, i, 0)),
            pl.BlockSpec((1, RB), lambda i: (0, i)),
            pl.BlockSpec((1, RB), lambda i: (0, i)),
            pl.BlockSpec((1, 128), lambda i: (0, 0)),
        ],
        out_shape=[
            jax.ShapeDtypeStruct((4, NP, 32), jnp.float32),
            jax.ShapeDtypeStruct((4, NP, 32), jnp.bfloat16),
            jax.ShapeDtypeStruct((1, NP), jnp.float32),
            jax.ShapeDtypeStruct((1, NP), jnp.float32),
            jax.ShapeDtypeStruct((1, 128), jnp.float32),
        ],
        scratch_shapes=[pltpu.SMEM((2,), jnp.float32)],
        compiler_params=pltpu.CompilerParams(
            dimension_semantics=("arbitrary",)),
    )(xpad, w, ats, atd)


def _tc_mid_body(acc_ref, dp_ref, as_ref, ad_ref, m_ref, xpt_ref, b_ref,
                 w2_ref, ats2_ref, atd2_ref,
                 xpt2_ref, xptb2_ref, as2_ref, ad2_ref, m2_ref, scr):
    i = pl.program_id(0)
    m = m_ref[0, 0]
    raw = as_ref[0] + ad_ref[0]
    pself = jnp.exp(_leaky(raw, 0.2) - m)
    den = dp_ref[0] + dp_ref[1] + pself + 1e-16
    acc = jnp.concatenate([acc_ref[b] for b in range(4)], axis=1)
    xp1 = jnp.concatenate([xpt_ref[b] for b in range(4)], axis=1)
    o = (acc + pself[:, None] * xp1) / den[:, None] + b_ref[...]
    h = _leaky(o, 0.01)
    xp2 = jnp.dot(h, w2_ref[...], preferred_element_type=jnp.float32)
    a_s = jnp.sum(xp2 * ats2_ref[...], axis=1)
    a_d = jnp.sum(xp2 * atd2_ref[...], axis=1)
    as2_ref[...] = a_s[None]
    ad2_ref[...] = a_d[None]
    for b in range(4):
        blkc = xp2[:, b * 32:(b + 1) * 32]
        xpt2_ref[b] = blkc
        xptb2_ref[b] = _interleave_halves(blkc).astype(jnp.bfloat16)
    ms = jnp.max(a_s)
    md = jnp.max(a_d)

    @pl.when(i == 0)
    def _():
        scr[0] = ms
        scr[1] = md

    @pl.when(i > 0)
    def _():
        scr[0] = jnp.maximum(scr[0], ms)
        scr[1] = jnp.maximum(scr[1], md)

    m2_ref[...] = jnp.full((1, 128), _leaky(scr[0] + scr[1], 0.2),
                           jnp.float32)


def _tc_mid(acc, dpart, a_s, a_d, mvec, xpt, bias, w2, ats2, atd2):
    return pl.pallas_call(
        _tc_mid_body,
        grid=(NBLK,),
        in_specs=[
            pl.BlockSpec((4, RB, 32), lambda i: (0, i, 0)),
            pl.BlockSpec((2, RB), lambda i: (0, i)),
            pl.BlockSpec((1, RB), lambda i: (0, i)),
            pl.BlockSpec((1, RB), lambda i: (0, i)),
            pl.BlockSpec((1, 128), lambda i: (0, 0)),
            pl.BlockSpec((4, RB, 32), lambda i: (0, i, 0)),
            pl.BlockSpec((1, H), lambda i: (0, 0)),
            pl.BlockSpec((H, H), lambda i: (0, 0)),
            pl.BlockSpec((1, H), lambda i: (0, 0)),
            pl.BlockSpec((1, H), lambda i: (0, 0)),
        ],
        out_specs=[
            pl.BlockSpec((4, RB, 32), lambda i: (0, i, 0)),
            pl.BlockSpec((4, RB, 32), lambda i: (0, i, 0)),
            pl.BlockSpec((1, RB), lambda i: (0, i)),
            pl.BlockSpec((1, RB), lambda i: (0, i)),
            pl.BlockSpec((1, 128), lambda i: (0, 0)),
        ],
        out_shape=[
            jax.ShapeDtypeStruct((4, NP, 32), jnp.float32),
            jax.ShapeDtypeStruct((4, NP, 32), jnp.bfloat16),
            jax.ShapeDtypeStruct((1, NP), jnp.float32),
            jax.ShapeDtypeStruct((1, NP), jnp.float32),
            jax.ShapeDtypeStruct((1, 128), jnp.float32),
        ],
        scratch_shapes=[pltpu.SMEM((2,), jnp.float32)],
        compiler_params=pltpu.CompilerParams(
            dimension_semantics=("arbitrary",),
            vmem_limit_bytes=100 * 1024 * 1024),
    )(acc, dpart, a_s, a_d, mvec, xpt, bias, w2, ats2, atd2)


def _tc_post_body(acc_ref, dp_ref, as_ref, ad_ref, m_ref, xpt_ref, b_ref,
                  out_ref):
    m = m_ref[0, 0]
    raw = as_ref[0] + ad_ref[0]
    pself = jnp.exp(_leaky(raw, 0.2) - m)
    den = dp_ref[0] + dp_ref[1] + pself + 1e-16
    acc = jnp.concatenate([acc_ref[b] for b in range(4)], axis=1)
    xp2 = jnp.concatenate([xpt_ref[b] for b in range(4)], axis=1)
    o = (acc + pself[:, None] * xp2) / den[:, None] + b_ref[...]
    out_ref[...] = _leaky(o, 0.01)


def _tc_post(acc, dpart, a_s, a_d, mvec, xpt, bias):
    return pl.pallas_call(
        _tc_post_body,
        grid=(NBLK,),
        in_specs=[
            pl.BlockSpec((4, RB, 32), lambda i: (0, i, 0)),
            pl.BlockSpec((2, RB), lambda i: (0, i)),
            pl.BlockSpec((1, RB), lambda i: (0, i)),
            pl.BlockSpec((1, RB), lambda i: (0, i)),
            pl.BlockSpec((1, 128), lambda i: (0, 0)),
            pl.BlockSpec((4, RB, 32), lambda i: (0, i, 0)),
            pl.BlockSpec((1, H), lambda i: (0, 0)),
        ],
        out_specs=pl.BlockSpec((RB, H), lambda i: (i, 0)),
        out_shape=jax.ShapeDtypeStruct((NP, H), jnp.float32),
        compiler_params=pltpu.CompilerParams(
            dimension_semantics=("arbitrary",)),
    )(acc, dpart, a_s, a_d, mvec, xpt, bias)


@functools.partial(
    pl.kernel,
    out_type=(jax.ShapeDtypeStruct((E,), jnp.float32),
              jax.ShapeDtypeStruct((2 * NP,), jnp.float32)),
    mesh=_mesh,
    scratch_types=[
        pltpu.VMEM((4, 128), jnp.int32),
        pltpu.VMEM((4, 128), jnp.int32),
        pltpu.VMEM((2, 128), jnp.float32),
        pltpu.VMEM((2, 128), jnp.float32),
        pltpu.VMEM((2, 128), jnp.float32),
        pltpu.VMEM((128,), jnp.float32),
        pltpu.VMEM((3136,), jnp.float32),
        pltpu.VMEM_SHARED((NP,), jnp.float32),
        pltpu.SemaphoreType.DMA((4,)),
        pltpu.SemaphoreType.DMA((2,)),
        pltpu.SemaphoreType.DMA((2,)),
    ])
def _sc_edge_p(src_hbm, dst_hbm, as_hbm, ad_hbm, m_hbm,
               p_hbm, dpart_hbm,
               srcb, dstb, asb, adb, pb, mb, zbuf, den_sh,
               semA, semB, semD):
    c = lax.axis_index("c")
    s = lax.axis_index("s")
    w = s * 2 + c

    def zz(j, carry):
        zbuf[pl.ds(j * 16, 16)] = jnp.zeros((16,), jnp.float32)
        return carry
    lax.fori_loop(0, 196, zz, 0)
    pltpu.sync_copy(zbuf.at[pl.ds(0, NPT)], den_sh.at[pl.ds(s * NPT, NPT)])
    plsc.subcore_barrier()

    pltpu.sync_copy(m_hbm, mb)
    mv = mb[pl.ds(0, 16)]

    cnt = Q1 + jnp.where(w < R1, 1, 0)

    def a_descs(g):
        par = g & 3
        e0 = (w + 32 * g) * 128
        return (
            pltpu.make_async_copy(src_hbm.at[pl.ds(e0, 128)], srcb.at[par],
                                  semA.at[par]),
            pltpu.make_async_copy(dst_hbm.at[pl.ds(e0, 128)], dstb.at[par],
                                  semA.at[par]),
        )

    def b_descs(g):
        return (
            pltpu.make_async_copy(as_hbm.at[srcb.at[g & 3]], asb.at[g & 1],
                                  semB.at[g & 1]),
            pltpu.make_async_copy(ad_hbm.at[dstb.at[g & 3]], adb.at[g & 1],
                                  semB.at[g & 1]),
        )

    def p_desc(g):
        e0 = (w + 32 * g) * 128
        return pltpu.make_async_copy(pb.at[g & 1], p_hbm.at[pl.ds(e0, 128)],
                                     semD.at[g & 1])

    def den_start(g):
        pltpu.sync_copy(pb.at[g & 1], den_sh.at[dstb.at[g & 3]], add=True)

    def do_p(g):
        pm2 = g & 1

        def cj(j, c2):
            sl = pl.ds(j * 16, 16)
            raw = asb[pm2, sl] + adb[pm2, sl]
            pb[pm2, sl] = jnp.exp(_leaky(raw, 0.2) - mv)
            return c2
        lax.fori_loop(0, 8, cj, 0)

    for d in a_descs(0):
        d.start()

    def body(g, carry):
        @pl.when(g + 1 < cnt)
        def _():
            for d in a_descs(g + 1):
                d.start()
        for d in a_descs(g):
            d.wait()

        @pl.when(g >= 2)
        def _():
            p_desc(g - 2).wait()
        for d in b_descs(g):
            d.start()

        @pl.when(g >= 1)
        def _():
            for d in b_descs(g - 1):
                d.wait()
            do_p(g - 1)
            p_desc(g - 1).start()
            den_start(g - 1)
        return carry
    lax.fori_loop(0, cnt, body, 0)

    gl = cnt - 1
    for d in b_descs(gl):
        d.wait()
    do_p(gl)
    p_desc(gl).start()
    den_start(gl)
    p_desc(gl - 1).wait()
    p_desc(gl).wait()

    plsc.subcore_barrier()
    cntf = QF + jnp.where(s < RF, 1, 0)

    def flush(t, carry):
        o = (s + 16 * t) * 128
        pltpu.sync_copy(den_sh.at[pl.ds(o, 128)],
                        dpart_hbm.at[pl.ds(c * NP + o, 128)])
        return carry
    lax.fori_loop(0, cntf, flush, 0)


@functools.partial(
    pl.kernel,
    out_type=jax.ShapeDtypeStruct((4, NP, 32), jnp.float32),
    mesh=_mesh,
    scratch_types=[
        pltpu.VMEM((4, 128), jnp.int32),
        pltpu.VMEM((4, 128), jnp.int32),
        pltpu.VMEM((4, 128), jnp.float32),
        pltpu.VMEM((2, 128, 32), jnp.bfloat16),
        pltpu.VMEM((2, 128, 32), jnp.float32),
        pltpu.VMEM_SHARED((NP, 32), jnp.float32),
        pltpu.SemaphoreType.DMA((4,)),
        pltpu.SemaphoreType.DMA((2,)),
        pltpu.SemaphoreType.DMA((2,)),
    ],
    compiler_params=pltpu.CompilerParams(use_tc_tiling_on_sc=False,
                                         needs_layout_passes=False))
def _sc_msg(src_hbm, dst_hbm, p_hbm, xpt_hbm, acc_hbm,
            srcb, dstb, pb, rowsb, rows, accum, semA, semB, semD):
    c = lax.axis_index("c")
    s = lax.axis_index("s")
    r0 = s * NPT
    cnt = Q2 + jnp.where(s < R2, 1, 0)

    def a_descs(g):
        par = g & 3
        e0 = (s + 16 * g) * 128
        return (
            pltpu.make_async_copy(src_hbm.at[pl.ds(e0, 128)], srcb.at[par],
                                  semA.at[par]),
            pltpu.make_async_copy(dst_hbm.at[pl.ds(e0, 128)], dstb.at[par],
                                  semA.at[par]),
            pltpu.make_async_copy(p_hbm.at[pl.ds(e0, 128)], pb.at[par],
                                  semA.at[par]),
        )

    def gat_desc(g):
        return pltpu.make_async_copy(xpt_hbm.at[srcb.at[g & 3]],
                                     rowsb.at[g & 1], semB.at[g & 1])

    def sct_start(g):
        pltpu.async_copy(rows.at[g & 1], accum.at[dstb.at[g & 3]],
                         semD.at[g & 1], add=True)

    def sct_wait(g):
        pltpu.make_async_copy(rows.at[g & 1], accum.at[dstb.at[g & 3]],
                              semD.at[g & 1]).wait()

    def do_scale(g):
        pm4 = g & 3
        pm2 = g & 1

        def scale(j, c2):
            pv = pb[pm4, pl.ds(j * 16, 16)]
            for t in range(16):
                k = j * 16 + t
                ps = pv[t]
                va, vb = plsc.unpack(rowsb[pm2, k, :],
                                     format=plsc.PackFormat.INTERLEAVED)
                rows[pm2, k, pl.ds(0, 16)] = va * ps
                rows[pm2, k, pl.ds(16, 16)] = vb * ps
            return c2
        lax.fori_loop(0, 8, scale, 0)

    for b in range(2):
        blk = 2 * b + c
        offv = jnp.zeros((16,), jnp.int32) + blk * NP

        def zr(j, carry):
            rows[0, j, pl.ds(0, 16)] = jnp.zeros((16,), jnp.float32)
            rows[0, j, pl.ds(16, 16)] = jnp.zeros((16,), jnp.float32)
            return carry
        lax.fori_loop(0, 128, zr, 0)
        for t in range(24):
            pltpu.sync_copy(rows.at[0], accum.at[pl.ds(r0 + t * 128, 128)])
        pltpu.sync_copy(rows.at[0, pl.ds(0, 56)],
                        accum.at[pl.ds(r0 + 24 * 128, 56)])
        plsc.subcore_barrier()

        for d in a_descs(0):
            d.start()

        def body(g, carry):
            @pl.when(g + 1 < cnt)
            def _():
                for d in a_descs(g + 1):
                    d.start()
            for d in a_descs(g):
                d.wait()
            par = g & 3
            for j in range(8):
                sl = pl.ds(j * 16, 16)
                srcb[par, sl] = srcb[par, sl] + offv

            @pl.when(g >= 2)
            def _():
                sct_wait(g - 2)
            gat_desc(g).start()

            @pl.when(g >= 1)
            def _():
                gat_desc(g - 1).wait()
                do_scale(g - 1)
                sct_start(g - 1)
            return carry
        lax.fori_loop(0, cnt, body, 0)

        gl = cnt - 1
        gat_desc(gl).wait()
        do_scale(gl)
        sct_start(gl)
        sct_wait(gl - 1)
        sct_wait(gl)

        plsc.subcore_barrier()
        pltpu.sync_copy(accum.at[pl.ds(r0, NPT)],
                        acc_hbm.at[blk, pl.ds(r0, NPT)])
        plsc.subcore_barrier()


def kernel(x, edge_index, W1, att_src1, att_dst1, b1,
           W2, att_src2, att_dst2, b2):
    src = edge_index[0].astype(jnp.int32)
    dst = edge_index[1].astype(jnp.int32)
    xpad = jnp.pad(x, ((0, NP - N), (0, 0)))

    xpt1, xptb1, as1, ad1, m1 = _tc_pre(xpad, W1,
                                 att_src1.reshape(1, H),
                                 att_dst1.reshape(1, H))
    p1, dpart1 = _sc_edge_p(src, dst, as1.reshape(NP), ad1.reshape(NP),
                            m1.reshape(128))
    acc1 = _sc_msg(src, dst, p1, xptb1.reshape(4 * NP, 32))
    xpt2, xptb2, as2, ad2, m2 = _tc_mid(acc1, dpart1.reshape(2, NP), as1,
                                        ad1,
                                 m1, xpt1, b1.reshape(1, H), W2,
                                 att_src2.reshape(1, H),
                                 att_dst2.reshape(1, H))
    p2, dpart2 = _sc_edge_p(src, dst, as2.reshape(NP), ad2.reshape(NP),
                            m2.reshape(128))
    acc2 = _sc_msg(src, dst, p2, xptb2.reshape(4 * NP, 32))
    out = _tc_post(acc2, dpart2.reshape(2, NP), as2, ad2, m2,
                   xpt2, b2.reshape(1, H))
    return out[:N]


# speedup vs baseline: 2.8352x; 2.8352x over previous
import functools

import jax
import jax.numpy as jnp
from jax import lax
from jax.experimental import pallas as pl
from jax.experimental.pallas import tpu as pltpu
from jax.experimental.pallas import tpu_sc as plsc

N = 50000
E = 800000
H = 128
NP = 50048
RB = 2176
NBLK = NP // RB
NPT = NP // 16
G = E // 128
Q1, R1 = divmod(G, 32)
Q2, R2 = divmod(G, 16)
NT = NP // 128
QF, RF = divmod(NT, 16)

_mesh = plsc.VectorSubcoreMesh(core_axis_name="c", subcore_axis_name="s",
                               num_cores=2, num_subcores=16)


def _leaky(v, slope):
    return jnp.where(v >= 0, v, v * slope)


def _tc_pre_body(x_ref, w_ref, ats_ref, atd_ref,
                 xpt_ref, as_ref, ad_ref, m_ref, scr):
    i = pl.program_id(0)
    xp = jnp.dot(x_ref[...], w_ref[...], preferred_element_type=jnp.float32)
    a_s = jnp.sum(xp * ats_ref[...], axis=1)
    a_d = jnp.sum(xp * atd_ref[...], axis=1)
    as_ref[...] = a_s[None]
    ad_ref[...] = a_d[None]
    for b in range(4):
        xpt_ref[b] = xp[:, b * 32:(b + 1) * 32]
    ms = jnp.max(a_s)
    md = jnp.max(a_d)

    @pl.when(i == 0)
    def _():
        scr[0] = ms
        scr[1] = md

    @pl.when(i > 0)
    def _():
        scr[0] = jnp.maximum(scr[0], ms)
        scr[1] = jnp.maximum(scr[1], md)

    m_ref[...] = jnp.full((1, 128), _leaky(scr[0] + scr[1], 0.2), jnp.float32)


def _tc_pre(xpad, w, ats, atd):
    f = xpad.shape[1]
    return pl.pallas_call(
        _tc_pre_body,
        grid=(NBLK,),
        in_specs=[
            pl.BlockSpec((RB, f), lambda i: (i, 0)),
            pl.BlockSpec((f, H), lambda i: (0, 0)),
            pl.BlockSpec((1, H), lambda i: (0, 0)),
            pl.BlockSpec((1, H), lambda i: (0, 0)),
        ],
        out_specs=[
            pl.BlockSpec((4, RB, 32), lambda i: (0, i, 0)),
            pl.BlockSpec((1, RB), lambda i: (0, i)),
            pl.BlockSpec((1, RB), lambda i: (0, i)),
            pl.BlockSpec((1, 128), lambda i: (0, 0)),
        ],
        out_shape=[
            jax.ShapeDtypeStruct((4, NP, 32), jnp.float32),
            jax.ShapeDtypeStruct((1, NP), jnp.float32),
            jax.ShapeDtypeStruct((1, NP), jnp.float32),
            jax.ShapeDtypeStruct((1, 128), jnp.float32),
        ],
        scratch_shapes=[pltpu.SMEM((2,), jnp.float32)],
        compiler_params=pltpu.CompilerParams(
            dimension_semantics=("arbitrary",)),
    )(xpad, w, ats, atd)


def _tc_mid_body(acc_ref, dp_ref, as_ref, ad_ref, m_ref, xpt_ref, b_ref,
                 w2_ref, ats2_ref, atd2_ref,
                 xpt2_ref, as2_ref, ad2_ref, m2_ref, scr):
    i = pl.program_id(0)
    m = m_ref[0, 0]
    raw = as_ref[0] + ad_ref[0]
    pself = jnp.exp(_leaky(raw, 0.2) - m)
    den = dp_ref[0] + dp_ref[1] + pself + 1e-16
    acc = jnp.concatenate([acc_ref[b] for b in range(4)], axis=1)
    xp1 = jnp.concatenate([xpt_ref[b] for b in range(4)], axis=1)
    o = (acc + pself[:, None] * xp1) / den[:, None] + b_ref[...]
    h = _leaky(o, 0.01)
    xp2 = jnp.dot(h, w2_ref[...], preferred_element_type=jnp.float32)
    a_s = jnp.sum(xp2 * ats2_ref[...], axis=1)
    a_d = jnp.sum(xp2 * atd2_ref[...], axis=1)
    as2_ref[...] = a_s[None]
    ad2_ref[...] = a_d[None]
    for b in range(4):
        xpt2_ref[b] = xp2[:, b * 32:(b + 1) * 32]
    ms = jnp.max(a_s)
    md = jnp.max(a_d)

    @pl.when(i == 0)
    def _():
        scr[0] = ms
        scr[1] = md

    @pl.when(i > 0)
    def _():
        scr[0] = jnp.maximum(scr[0], ms)
        scr[1] = jnp.maximum(scr[1], md)

    m2_ref[...] = jnp.full((1, 128), _leaky(scr[0] + scr[1], 0.2),
                           jnp.float32)


def _tc_mid(acc, dpart, a_s, a_d, mvec, xpt, bias, w2, ats2, atd2):
    return pl.pallas_call(
        _tc_mid_body,
        grid=(NBLK,),
        in_specs=[
            pl.BlockSpec((4, RB, 32), lambda i: (0, i, 0)),
            pl.BlockSpec((2, RB), lambda i: (0, i)),
            pl.BlockSpec((1, RB), lambda i: (0, i)),
            pl.BlockSpec((1, RB), lambda i: (0, i)),
            pl.BlockSpec((1, 128), lambda i: (0, 0)),
            pl.BlockSpec((4, RB, 32), lambda i: (0, i, 0)),
            pl.BlockSpec((1, H), lambda i: (0, 0)),
            pl.BlockSpec((H, H), lambda i: (0, 0)),
            pl.BlockSpec((1, H), lambda i: (0, 0)),
            pl.BlockSpec((1, H), lambda i: (0, 0)),
        ],
        out_specs=[
            pl.BlockSpec((4, RB, 32), lambda i: (0, i, 0)),
            pl.BlockSpec((1, RB), lambda i: (0, i)),
            pl.BlockSpec((1, RB), lambda i: (0, i)),
            pl.BlockSpec((1, 128), lambda i: (0, 0)),
        ],
        out_shape=[
            jax.ShapeDtypeStruct((4, NP, 32), jnp.float32),
            jax.ShapeDtypeStruct((1, NP), jnp.float32),
            jax.ShapeDtypeStruct((1, NP), jnp.float32),
            jax.ShapeDtypeStruct((1, 128), jnp.float32),
        ],
        scratch_shapes=[pltpu.SMEM((2,), jnp.float32)],
        compiler_params=pltpu.CompilerParams(
            dimension_semantics=("arbitrary",),
            vmem_limit_bytes=100 * 1024 * 1024),
    )(acc, dpart, a_s, a_d, mvec, xpt, bias, w2, ats2, atd2)


def _tc_post_body(acc_ref, dp_ref, as_ref, ad_ref, m_ref, xpt_ref, b_ref,
                  out_ref):
    m = m_ref[0, 0]
    raw = as_ref[0] + ad_ref[0]
    pself = jnp.exp(_leaky(raw, 0.2) - m)
    den = dp_ref[0] + dp_ref[1] + pself + 1e-16
    acc = jnp.concatenate([acc_ref[b] for b in range(4)], axis=1)
    xp2 = jnp.concatenate([xpt_ref[b] for b in range(4)], axis=1)
    o = (acc + pself[:, None] * xp2) / den[:, None] + b_ref[...]
    out_ref[...] = _leaky(o, 0.01)


def _tc_post(acc, dpart, a_s, a_d, mvec, xpt, bias):
    return pl.pallas_call(
        _tc_post_body,
        grid=(NBLK,),
        in_specs=[
            pl.BlockSpec((4, RB, 32), lambda i: (0, i, 0)),
            pl.BlockSpec((2, RB), lambda i: (0, i)),
            pl.BlockSpec((1, RB), lambda i: (0, i)),
            pl.BlockSpec((1, RB), lambda i: (0, i)),
            pl.BlockSpec((1, 128), lambda i: (0, 0)),
            pl.BlockSpec((4, RB, 32), lambda i: (0, i, 0)),
            pl.BlockSpec((1, H), lambda i: (0, 0)),
        ],
        out_specs=pl.BlockSpec((RB, H), lambda i: (i, 0)),
        out_shape=jax.ShapeDtypeStruct((NP, H), jnp.float32),
        compiler_params=pltpu.CompilerParams(
            dimension_semantics=("arbitrary",)),
    )(acc, dpart, a_s, a_d, mvec, xpt, bias)


@functools.partial(
    pl.kernel,
    out_type=(jax.ShapeDtypeStruct((E,), jnp.float32),
              jax.ShapeDtypeStruct((2 * NP,), jnp.float32)),
    mesh=_mesh,
    scratch_types=[
        pltpu.VMEM((4, 128), jnp.int32),
        pltpu.VMEM((4, 128), jnp.int32),
        pltpu.VMEM((2, 128), jnp.float32),
        pltpu.VMEM((2, 128), jnp.float32),
        pltpu.VMEM((2, 128), jnp.float32),
        pltpu.VMEM((128,), jnp.float32),
        pltpu.VMEM((3136,), jnp.float32),
        pltpu.VMEM_SHARED((NP,), jnp.float32),
        pltpu.SemaphoreType.DMA((4,)),
        pltpu.SemaphoreType.DMA((2,)),
        pltpu.SemaphoreType.DMA((2,)),
    ])
def _sc_edge_p(src_hbm, dst_hbm, as_hbm, ad_hbm, m_hbm,
               p_hbm, dpart_hbm,
               srcb, dstb, asb, adb, pb, mb, zbuf, den_sh,
               semA, semB, semD):
    c = lax.axis_index("c")
    s = lax.axis_index("s")
    w = s * 2 + c

    def zz(j, carry):
        zbuf[pl.ds(j * 16, 16)] = jnp.zeros((16,), jnp.float32)
        return carry
    lax.fori_loop(0, 196, zz, 0)
    pltpu.sync_copy(zbuf.at[pl.ds(0, NPT)], den_sh.at[pl.ds(s * NPT, NPT)])
    plsc.subcore_barrier()

    pltpu.sync_copy(m_hbm, mb)
    mv = mb[pl.ds(0, 16)]

    cnt = Q1 + jnp.where(w < R1, 1, 0)

    def a_descs(g):
        par = g & 3
        e0 = (w + 32 * g) * 128
        return (
            pltpu.make_async_copy(src_hbm.at[pl.ds(e0, 128)], srcb.at[par],
                                  semA.at[par]),
            pltpu.make_async_copy(dst_hbm.at[pl.ds(e0, 128)], dstb.at[par],
                                  semA.at[par]),
        )

    def b_descs(g):
        return (
            pltpu.make_async_copy(as_hbm.at[srcb.at[g & 3]], asb.at[g & 1],
                                  semB.at[g & 1]),
            pltpu.make_async_copy(ad_hbm.at[dstb.at[g & 3]], adb.at[g & 1],
                                  semB.at[g & 1]),
        )

    def p_desc(g):
        e0 = (w + 32 * g) * 128
        return pltpu.make_async_copy(pb.at[g & 1], p_hbm.at[pl.ds(e0, 128)],
                                     semD.at[g & 1])

    def den_start(g):
        pltpu.sync_copy(pb.at[g & 1], den_sh.at[dstb.at[g & 3]], add=True)

    def do_p(g):
        pm2 = g & 1

        def cj(j, c2):
            sl = pl.ds(j * 16, 16)
            raw = asb[pm2, sl] + adb[pm2, sl]
            pb[pm2, sl] = jnp.exp(_leaky(raw, 0.2) - mv)
            return c2
        lax.fori_loop(0, 8, cj, 0)

    for d in a_descs(0):
        d.start()

    def body(g, carry):
        @pl.when(g + 1 < cnt)
        def _():
            for d in a_descs(g + 1):
                d.start()
        for d in a_descs(g):
            d.wait()

        @pl.when(g >= 2)
        def _():
            p_desc(g - 2).wait()
        for d in b_descs(g):
            d.start()

        @pl.when(g >= 1)
        def _():
            for d in b_descs(g - 1):
                d.wait()
            do_p(g - 1)
            p_desc(g - 1).start()
            den_start(g - 1)
        return carry
    lax.fori_loop(0, cnt, body, 0)

    gl = cnt - 1
    for d in b_descs(gl):
        d.wait()
    do_p(gl)
    p_desc(gl).start()
    den_start(gl)
    p_desc(gl - 1).wait()
    p_desc(gl).wait()

    plsc.subcore_barrier()
    cntf = QF + jnp.where(s < RF, 1, 0)

    def flush(t, carry):
        o = (s + 16 * t) * 128
        pltpu.sync_copy(den_sh.at[pl.ds(o, 128)],
                        dpart_hbm.at[pl.ds(c * NP + o, 128)])
        return carry
    lax.fori_loop(0, cntf, flush, 0)


@functools.partial(
    pl.kernel,
    out_type=jax.ShapeDtypeStruct((4, NP, 32), jnp.float32),
    mesh=_mesh,
    scratch_types=[
        pltpu.VMEM((4, 128), jnp.int32),
        pltpu.VMEM((4, 128), jnp.int32),
        pltpu.VMEM((4, 128), jnp.float32),
        pltpu.VMEM((2, 128, 32), jnp.float32),
        pltpu.VMEM_SHARED((NP, 32), jnp.float32),
        pltpu.SemaphoreType.DMA((4,)),
        pltpu.SemaphoreType.DMA((2,)),
        pltpu.SemaphoreType.DMA((2,)),
    ],
    compiler_params=pltpu.CompilerParams(use_tc_tiling_on_sc=False))
def _sc_msg(src_hbm, dst_hbm, p_hbm, xpt_hbm, acc_hbm,
            srcb, dstb, pb, rows, accum, semA, semB, semD):
    c = lax.axis_index("c")
    s = lax.axis_index("s")
    r0 = s * NPT
    cnt = Q2 + jnp.where(s < R2, 1, 0)

    def a_descs(g):
        par = g & 3
        e0 = (s + 16 * g) * 128
        return (
            pltpu.make_async_copy(src_hbm.at[pl.ds(e0, 128)], srcb.at[par],
                                  semA.at[par]),
            pltpu.make_async_copy(dst_hbm.at[pl.ds(e0, 128)], dstb.at[par],
                                  semA.at[par]),
            pltpu.make_async_copy(p_hbm.at[pl.ds(e0, 128)], pb.at[par],
                                  semA.at[par]),
        )

    def gat_desc(g):
        return pltpu.make_async_copy(xpt_hbm.at[srcb.at[g & 3]],
                                     rows.at[g & 1], semB.at[g & 1])

    def sct_start(g):
        pltpu.async_copy(rows.at[g & 1], accum.at[dstb.at[g & 3]],
                         semD.at[g & 1], add=True)

    def sct_wait(g):
        pltpu.make_async_copy(rows.at[g & 1], accum.at[dstb.at[g & 3]],
                              semD.at[g & 1]).wait()

    def do_scale(g):
        pm4 = g & 3
        pm2 = g & 1

        def scale(j, c2):
            pv = pb[pm4, pl.ds(j * 16, 16)]
            for t in range(16):
                k = j * 16 + t
                ps = pv[t]
                rows[pm2, k, pl.ds(0, 16)] = rows[pm2, k, pl.ds(0, 16)] * ps
                rows[pm2, k, pl.ds(16, 16)] = rows[pm2, k, pl.ds(16, 16)] * ps
            return c2
        lax.fori_loop(0, 8, scale, 0)

    for b in range(2):
        blk = 2 * b + c
        offv = jnp.zeros((16,), jnp.int32) + blk * NP

        def zr(j, carry):
            rows[0, j, pl.ds(0, 16)] = jnp.zeros((16,), jnp.float32)
            rows[0, j, pl.ds(16, 16)] = jnp.zeros((16,), jnp.float32)
            return carry
        lax.fori_loop(0, 128, zr, 0)
        for t in range(24):
            pltpu.sync_copy(rows.at[0], accum.at[pl.ds(r0 + t * 128, 128)])
        pltpu.sync_copy(rows.at[0, pl.ds(0, 56)],
                        accum.at[pl.ds(r0 + 24 * 128, 56)])
        plsc.subcore_barrier()

        for d in a_descs(0):
            d.start()

        def body(g, carry):
            @pl.when(g + 1 < cnt)
            def _():
                for d in a_descs(g + 1):
                    d.start()
            for d in a_descs(g):
                d.wait()
            par = g & 3
            for j in range(8):
                sl = pl.ds(j * 16, 16)
                srcb[par, sl] = srcb[par, sl] + offv

            @pl.when(g >= 2)
            def _():
                sct_wait(g - 2)
            gat_desc(g).start()

            @pl.when(g >= 1)
            def _():
                gat_desc(g - 1).wait()
                do_scale(g - 1)
                sct_start(g - 1)
            return carry
        lax.fori_loop(0, cnt, body, 0)

        gl = cnt - 1
        gat_desc(gl).wait()
        do_scale(gl)
        sct_start(gl)
        sct_wait(gl - 1)
        sct_wait(gl)

        plsc.subcore_barrier()
        pltpu.sync_copy(accum.at[pl.ds(r0, NPT)],
                        acc_hbm.at[blk, pl.ds(r0, NPT)])
        plsc.subcore_barrier()


def kernel(x, edge_index, W1, att_src1, att_dst1, b1,
           W2, att_src2, att_dst2, b2):
    src = edge_index[0].astype(jnp.int32)
    dst = edge_index[1].astype(jnp.int32)
    xpad = jnp.pad(x, ((0, NP - N), (0, 0)))

    xpt1, as1, ad1, m1 = _tc_pre(xpad, W1,
                                 att_src1.reshape(1, H),
                                 att_dst1.reshape(1, H))
    p1, dpart1 = _sc_edge_p(src, dst, as1.reshape(NP), ad1.reshape(NP),
                            m1.reshape(128))
    acc1 = _sc_msg(src, dst, p1, xpt1.reshape(4 * NP, 32))
    xpt2, as2, ad2, m2 = _tc_mid(acc1, dpart1.reshape(2, NP), as1, ad1,
                                 m1, xpt1, b1.reshape(1, H), W2,
                                 att_src2.reshape(1, H),
                                 att_dst2.reshape(1, H))
    p2, dpart2 = _sc_edge_p(src, dst, as2.reshape(NP), ad2.reshape(NP),
                            m2.reshape(128))
    acc2 = _sc_msg(src, dst, p2, xpt2.reshape(4 * NP, 32))
    out = _tc_post(acc2, dpart2.reshape(2, NP), as2, ad2, m2,
                   xpt2, b2.reshape(1, H))
    return out[:N]
